# Initial kernel scaffold; baseline (speedup 1.0000x reference)
import jax
import jax.numpy as jnp
from jax.experimental import pallas as pl


def kernel(feat, offset, ctx_ae, ctx_sinr, W_ctx0, b_ctx0, W_ctx1, b_ctx1, W1, b1, ln_g, ln_b, W2, b2):
    raise NotImplementedError("write your pallas kernel here")


# speedup vs baseline: 5.1897x; 5.1897x over previous
import functools

import jax
import jax.numpy as jnp
from jax import lax
from jax.experimental import pallas as pl
from jax.experimental.pallas import tpu as pltpu
from jax.experimental.pallas import tpu_sc as plsc

B = 16
N = 32768
D = 512
CIN = 64
CE = 256
NC = 13
HID = 512

CHUNK = 64
NCH = N // CHUNK
NW = 32
CPW = NCH // NW
LANES = 16
NVR = D // LANES


def _sc_partial_sums(feat, offset):
    mesh = plsc.VectorSubcoreMesh(core_axis_name="c", subcore_axis_name="s")

    @functools.partial(
        pl.kernel,
        mesh=mesh,
        compiler_params=pltpu.CompilerParams(needs_layout_passes=False),
        out_type=(
            jax.ShapeDtypeStruct((NCH, D), jnp.float32),
            jax.ShapeDtypeStruct((B, D), jnp.float32),
        ),
        scratch_types=[
            pltpu.VMEM((CHUNK, D), jnp.float32),
            pltpu.VMEM((CHUNK, D), jnp.float32),
            pltpu.VMEM((CPW, D), jnp.float32),
            pltpu.VMEM((1, D), jnp.float32),
            pltpu.VMEM((LANES,), jnp.int32),
            pltpu.VMEM((CHUNK,), jnp.int32),
            pltpu.SemaphoreType.DMA,
            pltpu.SemaphoreType.DMA,
        ],
    )
    def body(feat_hbm, off_hbm, chunks_out, corr_out,
             buf0, buf1, accb, rowb, offs_v, idx_v, sem0, sem1):
        cid = lax.axis_index("c")
        sid = lax.axis_index("s")
        wid = sid * 2 + cid
        base_chunk = wid * CPW
        bufs = (buf0, buf1)
        sems = (sem0, sem1)
        zero = jnp.zeros((LANES,), jnp.float32)

        copies = [pltpu.async_copy(
            feat_hbm.at[pl.ds(pl.multiple_of(base_chunk * CHUNK, CHUNK), CHUNK)],
            buf0, sem0)]
        for c in range(CPW):
            if c + 1 < CPW:
                copies.append(pltpu.async_copy(
                    feat_hbm.at[pl.ds(pl.multiple_of(
                        (base_chunk + c + 1) * CHUNK, CHUNK), CHUNK)],
                    bufs[(c + 1) % 2], sems[(c + 1) % 2]))
            copies[c].wait()
            buf = bufs[c % 2]

            def rbody(r, acc):
                return tuple(acc[j] + buf[r, pl.ds(j * LANES, LANES)]
                             for j in range(NVR))

            acc = lax.fori_loop(0, CHUNK, rbody, (zero,) * NVR)
            for j in range(NVR):
                accb[c, pl.ds(j * LANES, LANES)] = acc[j]
        pltpu.sync_copy(accb, chunks_out.at[pl.ds(base_chunk, CPW)])

        @pl.when(wid < B)
        def _():
            pltpu.sync_copy(off_hbm, offs_v)
            lane = lax.iota(jnp.int32, LANES)
            o_b = plsc.load_gather(offs_v, [lane * 0 + wid])
            fc = o_b >> 6
            m = o_b - (fc << 6)
            base = fc << 6
            for j in range(CHUNK // LANES):
                idx_v[pl.ds(j * LANES, LANES)] = jnp.minimum(
                    base + j * LANES + lane, N - 1)
            pltpu.async_copy(feat_hbm.at[idx_v], buf0, sem0).wait()

            def cbody(r, acc):
                keep = r < m
                return tuple(
                    acc[j] + jnp.where(keep,
                                       buf0[r, pl.ds(j * LANES, LANES)], 0.0)
                    for j in range(NVR))

            cacc = lax.fori_loop(0, CHUNK, cbody, (zero,) * NVR)
            for j in range(NVR):
                rowb[0, pl.ds(j * LANES, LANES)] = cacc[j]
            pltpu.sync_copy(rowb, corr_out.at[pl.ds(wid, 1)])

    return body(feat, offset)


def _tc_head(chunk_sums, corr, offset2d, ctx_ae, ctx_sinr,
             W_ctx0, b_ctx0, W_ctx1, b_ctx1,
             W1a, W1b, W1c, b1, ln_g, ln_b, W2, b2):
    def body(cs_ref, corr_ref, off_ref, ae_ref, sinr_ref,
             w0_ref, bb0_ref, w1_ref, bb1_ref,
             W1a_ref, W1b_ref, W1c_ref, b1_ref, g_ref, be_ref,
             W2_ref, b2_ref, out_ref):
        o = off_ref[...]
        zi = jnp.zeros((1, 1), jnp.int32)
        s = jnp.concatenate([zi, o[:B - 1, :]], axis=0)
        fc_e = o >> 6
        fc_s = s >> 6
        ciota = lax.broadcasted_iota(jnp.int32, (B, NCH), 1)
        sel = jnp.logical_and(ciota >= fc_s, ciota < fc_e).astype(jnp.float32)
        tree = jnp.dot(sel, cs_ref[...], preferred_element_type=jnp.float32,
                       precision=lax.Precision.HIGHEST)
        ce = corr_ref[...]
        zf = jnp.zeros((1, D), jnp.float32)
        cs_prev = jnp.concatenate([zf, ce[:B - 1, :]], axis=0)
        tree = tree + ce - cs_prev
        counts = (o - s).astype(jnp.float32)
        tree = tree / jnp.maximum(counts, 1.0)

        c0 = jnp.dot(ae_ref[...], w0_ref[...],
                     preferred_element_type=jnp.float32) + bb0_ref[...]
        c1 = jnp.dot(sinr_ref[...], w1_ref[...],
                     preferred_element_type=jnp.float32) + bb1_ref[...]
        h = (jnp.dot(tree, W1a_ref[...], preferred_element_type=jnp.float32)
             + jnp.dot(c0, W1b_ref[...], preferred_element_type=jnp.float32)
             + jnp.dot(c1, W1c_ref[...], preferred_element_type=jnp.float32)
             + b1_ref[...])
        mu = jnp.mean(h, axis=-1, keepdims=True)
        var = jnp.mean((h - mu) ** 2, axis=-1, keepdims=True)
        h = (h - mu) * lax.rsqrt(var + 1e-5) * g_ref[...] + be_ref[...]
        h = jnp.maximum(h, 0.0)
        out_ref[...] = jnp.dot(h, W2_ref[...],
                               preferred_element_type=jnp.float32) + b2_ref[...]

    return pl.pallas_call(
        body,
        out_shape=jax.ShapeDtypeStruct((B, NC), jnp.float32),
    )(chunk_sums, corr, offset2d, ctx_ae, ctx_sinr,
      W_ctx0, b_ctx0, W_ctx1, b_ctx1,
      W1a, W1b, W1c, b1, ln_g, ln_b, W2, b2)


def kernel(feat, offset, ctx_ae, ctx_sinr, W_ctx0, b_ctx0, W_ctx1, b_ctx1,
           W1, b1, ln_g, ln_b, W2, b2):
    chunk_sums, corr = _sc_partial_sums(feat, offset)
    return _tc_head(
        chunk_sums, corr, offset.reshape(B, 1), ctx_ae, ctx_sinr,
        W_ctx0, b_ctx0.reshape(1, CE), W_ctx1, b_ctx1.reshape(1, CE),
        W1[:D, :], W1[D:D + CE, :], W1[D + CE:, :],
        b1.reshape(1, HID), ln_g.reshape(1, HID), ln_b.reshape(1, HID),
        W2, b2.reshape(1, NC))
